# baseline (speedup 1.0000x reference)
import functools

import jax
import jax.numpy as jnp
from jax import lax
from jax.experimental import pallas as pl
from jax.experimental.pallas import tpu as pltpu
from jax.experimental.pallas import tpu_sc as plsc

_NC = 2
_NS = 16
_NW = _NC * _NS
_L = 16

_D = 64
_N_ROWS = 4096 * 200
_ROWS_PER_W = _N_ROWS // _NW
_CHUNK = 1024
_GSZ = 128
_GPC = _CHUNK // _GSZ
_N_CHUNKS = _ROWS_PER_W // _CHUNK
_EPS = 1e-6


def _ln_rows(rows_v, gamma_v, beta_v, n_rows):
    gs = [gamma_v[pl.ds(k * _L, _L)] for k in range(4)]
    bs = [beta_v[pl.ds(k * _L, _L)] for k in range(4)]

    def row_body(i, _):
        x = [rows_v[i, pl.ds(k * _L, _L)] for k in range(4)]
        s = (x[0] + x[1]) + (x[2] + x[3])
        sq = (x[0] * x[0] + x[1] * x[1]) + (x[2] * x[2] + x[3] * x[3])
        mean = jnp.sum(s) * (1.0 / _D)
        ex2 = jnp.sum(sq) * (1.0 / _D)
        var = ex2 - mean * mean
        r = var + _EPS
        ib = lax.bitcast_convert_type(r, jnp.int32)
        ib = 0x5F3759DF - lax.shift_right_logical(ib, 1)
        y = lax.bitcast_convert_type(ib, jnp.float32)
        y = y * (1.5 - 0.5 * r * y * y)
        y = y * (1.5 - 0.5 * r * y * y)
        y = y * (1.5 - 0.5 * r * y * y)
        for k in range(4):
            rows_v[i, pl.ds(k * _L, _L)] = (x[k] - mean) * (y * gs[k]) + bs[k]
        return 0

    lax.fori_loop(0, n_rows, row_body, 0)


def _body(idx_hbm, table_hbm, gamma_hbm, beta_hbm, out_hbm,
          idx_v, rows_v, gamma_v, beta_v, gsem):
    wid = lax.axis_index("s") * _NC + lax.axis_index("c")
    row_base = wid * _ROWS_PER_W
    pltpu.sync_copy(gamma_hbm, gamma_v)
    pltpu.sync_copy(beta_hbm, beta_v)

    def chunk_body(c, _):
        base = row_base + c * _CHUNK
        irow = pl.multiple_of(base // _GSZ, 8)
        pltpu.sync_copy(idx_hbm.at[pl.ds(irow, _GPC)], idx_v)
        cps = [
            pltpu.async_copy(
                table_hbm.at[idx_v.at[j]],
                rows_v.at[pl.ds(j * _GSZ, _GSZ)],
                gsem,
            )
            for j in range(_GPC)
        ]
        for cp in cps:
            cp.wait()
        _ln_rows(rows_v, gamma_v, beta_v, _CHUNK)
        pltpu.sync_copy(rows_v, out_hbm.at[pl.ds(base, _CHUNK)])
        return 0

    lax.fori_loop(0, _N_CHUNKS, chunk_body, 0)


@jax.jit
def _sc_lookup_ln(idx2d, table, gamma, beta):
    mesh = plsc.VectorSubcoreMesh(core_axis_name="c", subcore_axis_name="s")
    f = pl.kernel(
        _body,
        out_type=jax.ShapeDtypeStruct((_N_ROWS, _D), jnp.float32),
        mesh=mesh,
        scratch_types=[
            pltpu.VMEM((_GPC, _GSZ), jnp.int32),
            pltpu.VMEM((_CHUNK, _D), jnp.float32),
            pltpu.VMEM((_D,), jnp.float32),
            pltpu.VMEM((_D,), jnp.float32),
            pltpu.SemaphoreType.DMA,
        ],
        compiler_params=pltpu.CompilerParams(
            needs_layout_passes=False, use_tc_tiling_on_sc=False),
    )
    return f(idx2d, table, gamma, beta)


def kernel(src, word_embedding, ln_gamma, ln_beta):
    idx2d = src.reshape(-1, _GSZ).astype(jnp.int32)
    out = _sc_lookup_ln(idx2d, word_embedding, ln_gamma, ln_beta)
    return out.reshape(src.shape + (_D,))


# speedup vs baseline: 1.0007x; 1.0007x over previous
import jax
import jax.numpy as jnp
from jax import lax
from jax.experimental import pallas as pl
from jax.experimental.pallas import tpu as pltpu
from jax.experimental.pallas import tpu_sc as plsc

_NC = 2
_NS = 16
_NW = _NC * _NS
_L = 16

_D = 64
_N_ROWS = 4096 * 200
_ROWS_PER_W = _N_ROWS // _NW
_CHUNK = 512
_GSZ = 128
_GPC = _CHUNK // _GSZ
_N_CHUNKS = _ROWS_PER_W // _CHUNK
_IDX_ROWS = _ROWS_PER_W // _GSZ
_GROUPS = _CHUNK // _L
_EPS = 1e-6


def _ln_chunk(rows_v, gamma_v, beta_v):
    gs = [gamma_v[pl.ds(k * _L, _L)] for k in range(4)]
    bs = [beta_v[pl.ds(k * _L, _L)] for k in range(4)]
    lanes = lax.iota(jnp.int32, _L)

    def group_body(g, _):
        rb = g * _L
        rid = rb + lanes
        s = None
        q = None
        for j in range(_D):
            cid = jnp.full((_L,), j, jnp.int32)
            x = plsc.load_gather(rows_v, [rid, cid])
            s = x if s is None else s + x
            q = x * x if q is None else q + x * x
        mean = s * (1.0 / _D)
        var = q * (1.0 / _D) - mean * mean
        r = var + _EPS
        ib = plsc.bitcast(r, jnp.int32)
        ib = 0x5F3759DF - lax.shift_right_logical(ib, 1)
        y = plsc.bitcast(ib, jnp.float32)
        y = y * (1.5 - 0.5 * r * y * y)
        y = y * (1.5 - 0.5 * r * y * y)
        y = y * (1.5 - 0.5 * r * y * y)
        for i in range(_L):
            m_i = mean[i]
            a_i = y[i]
            row = rb + i
            for k in range(4):
                xk = rows_v[row, pl.ds(k * _L, _L)]
                rows_v[row, pl.ds(k * _L, _L)] = (xk - m_i) * (a_i * gs[k]) + bs[k]
        return 0

    lax.fori_loop(0, _GROUPS, group_body, 0)


def _body(idx_hbm, table_hbm, gamma_hbm, beta_hbm, out_hbm,
          idx_v, rows0, rows1, gamma_v, beta_v, gsem0, gsem1, wsem0, wsem1):
    wid = lax.axis_index("s") * _NC + lax.axis_index("c")
    row_base = wid * _ROWS_PER_W
    pltpu.sync_copy(gamma_hbm, gamma_v)
    pltpu.sync_copy(beta_hbm, beta_v)
    pltpu.sync_copy(idx_hbm.at[pl.ds(wid * _IDX_ROWS, _IDX_ROWS)], idx_v)

    rows = (rows0, rows1)
    gsems = (gsem0, gsem1)
    wsems = (wsem0, wsem1)

    def fire_gather(c, buf, sem):
        for j in range(_GPC):
            pltpu.async_copy(
                table_hbm.at[idx_v.at[c * _GPC + j]],
                buf.at[pl.ds(j * _GSZ, _GSZ)],
                sem,
            )

    def drain(buf, sem):
        pltpu.make_async_copy(table_hbm.at[pl.ds(0, _CHUNK)], buf, sem).wait()

    fire_gather(0, rows0, gsem0)

    def super_body(sc, _):
        for b in range(2):
            c = sc * 2 + b
            nb = 1 - b

            @pl.when(c + 1 < _N_CHUNKS)
            def _prefetch():
                @pl.when(c >= 1)
                def _recycle():
                    drain(rows[nb], wsems[nb])
                fire_gather(c + 1, rows[nb], gsems[nb])

            drain(rows[b], gsems[b])
            _ln_chunk(rows[b], gamma_v, beta_v)
            pltpu.async_copy(
                rows[b],
                out_hbm.at[pl.ds(row_base + c * _CHUNK, _CHUNK)],
                wsems[b],
            )
        return 0

    lax.fori_loop(0, _N_CHUNKS // 2, super_body, 0)
    drain(rows0, wsem0)
    drain(rows1, wsem1)


@jax.jit
def _sc_lookup_ln(idx2d, table, gamma, beta):
    mesh = plsc.VectorSubcoreMesh(core_axis_name="c", subcore_axis_name="s")
    f = pl.kernel(
        _body,
        out_type=jax.ShapeDtypeStruct((_N_ROWS, _D), jnp.float32),
        mesh=mesh,
        scratch_types=[
            pltpu.VMEM((_IDX_ROWS, _GSZ), jnp.int32),
            pltpu.VMEM((_CHUNK, _D), jnp.float32),
            pltpu.VMEM((_CHUNK, _D), jnp.float32),
            pltpu.VMEM((_D,), jnp.float32),
            pltpu.VMEM((_D,), jnp.float32),
            pltpu.SemaphoreType.DMA,
            pltpu.SemaphoreType.DMA,
            pltpu.SemaphoreType.DMA,
            pltpu.SemaphoreType.DMA,
        ],
        compiler_params=pltpu.CompilerParams(
            needs_layout_passes=False, use_tc_tiling_on_sc=False),
    )
    return f(idx2d, table, gamma, beta)


def kernel(src, word_embedding, ln_gamma, ln_beta):
    idx2d = src.reshape(-1, _GSZ).astype(jnp.int32)
    out = _sc_lookup_ln(idx2d, word_embedding, ln_gamma, ln_beta)
    return out.reshape(src.shape + (_D,))


# speedup vs baseline: 1.6680x; 1.6668x over previous
import jax
import jax.numpy as jnp
from jax import lax
from jax.experimental import pallas as pl
from jax.experimental.pallas import tpu as pltpu
from jax.experimental.pallas import tpu_sc as plsc

_NC = 2
_NS = 16
_NW = _NC * _NS
_L = 16

_D = 64
_B = 4096
_S = 200
_SRC_PER_W = _B // _NW
_SPC = 2
_CHUNK = _SPC * _S
_N_CHUNKS = _SRC_PER_W // _SPC
_GROUPS = _CHUNK // _L
_EPS = 1e-6


def _ln_chunk(rows_v, gamma_v, beta_v, perms):
    gs = [gamma_v[pl.ds(k * _L, _L)] for k in range(4)]
    bs = [beta_v[pl.ds(k * _L, _L)] for k in range(4)]

    def group_body(g, _):
        rb = g * _L
        for i in range(_L):
            row = rb + i
            x = [rows_v[row, pl.ds(k * _L, _L)] for k in range(4)]
            s = (x[0] + x[1]) + (x[2] + x[3])
            q = (x[0] * x[0] + x[1] * x[1]) + (x[2] * x[2] + x[3] * x[3])
            for p in perms:
                s = s + s.at[p].get(mode="promise_in_bounds")
                q = q + q.at[p].get(mode="promise_in_bounds")
            mean = s * (1.0 / _D)
            var = q * (1.0 / _D) - mean * mean
            r = var + _EPS
            ib = plsc.bitcast(r, jnp.int32)
            ib = 0x5F3759DF - lax.shift_right_logical(ib, 1)
            y = plsc.bitcast(ib, jnp.float32)
            y = y * (1.5 - 0.5 * r * y * y)
            y = y * (1.5 - 0.5 * r * y * y)
            for k in range(4):
                rows_v[row, pl.ds(k * _L, _L)] = (x[k] - mean) * (y * gs[k]) + bs[k]
        return 0

    lax.fori_loop(0, _GROUPS, group_body, 0)


def _body(src_hbm, table_hbm, gamma_hbm, beta_hbm, out_hbm,
          idx_v, rows0, rows1, gamma_v, beta_v, gsem0, gsem1, wsem0, wsem1):
    wid = lax.axis_index("s") * _NC + lax.axis_index("c")
    src_base = wid * _SRC_PER_W
    pltpu.sync_copy(gamma_hbm, gamma_v)
    pltpu.sync_copy(beta_hbm, beta_v)
    pltpu.sync_copy(src_hbm.at[pl.ds(src_base, _SRC_PER_W)], idx_v)

    rows = (rows0, rows1)
    gsems = (gsem0, gsem1)
    wsems = (wsem0, wsem1)
    perms = [lax.iota(jnp.int32, _L) ^ sh for sh in (8, 4, 2, 1)]

    def fire_gather(c, buf, sem):
        for sr in range(_SPC):
            r = c * _SPC + sr
            pltpu.async_copy(table_hbm.at[idx_v.at[r, pl.ds(0, 128)]],
                             buf.at[pl.ds(sr * _S, 128)], sem)
            pltpu.async_copy(table_hbm.at[idx_v.at[r, pl.ds(128, _S - 128)]],
                             buf.at[pl.ds(sr * _S + 128, _S - 128)], sem)

    def drain(buf, sem):
        pltpu.make_async_copy(table_hbm.at[pl.ds(0, _CHUNK)], buf, sem).wait()

    fire_gather(0, rows0, gsem0)

    def super_body(sc, _):
        for b in range(2):
            c = sc * 2 + b
            nb = 1 - b

            @pl.when(c + 1 < _N_CHUNKS)
            def _prefetch():
                @pl.when(c >= 1)
                def _recycle():
                    drain(rows[nb], wsems[nb])
                fire_gather(c + 1, rows[nb], gsems[nb])

            drain(rows[b], gsems[b])
            _ln_chunk(rows[b], gamma_v, beta_v, perms)
            for sr in range(_SPC):
                pltpu.async_copy(rows[b].at[pl.ds(sr * _S, _S)],
                                 out_hbm.at[src_base + c * _SPC + sr],
                                 wsems[b])
        return 0

    lax.fori_loop(0, _N_CHUNKS // 2, super_body, 0)
    drain(rows0, wsem0)
    drain(rows1, wsem1)


@jax.jit
def _sc_lookup_ln(src, table, gamma, beta):
    mesh = plsc.VectorSubcoreMesh(core_axis_name="c", subcore_axis_name="s")
    f = pl.kernel(
        _body,
        out_type=jax.ShapeDtypeStruct((_B, _S, _D), jnp.float32),
        mesh=mesh,
        scratch_types=[
            pltpu.VMEM((_SRC_PER_W, _S), jnp.int32),
            pltpu.VMEM((_CHUNK, _D), jnp.float32),
            pltpu.VMEM((_CHUNK, _D), jnp.float32),
            pltpu.VMEM((_D,), jnp.float32),
            pltpu.VMEM((_D,), jnp.float32),
            pltpu.SemaphoreType.DMA,
            pltpu.SemaphoreType.DMA,
            pltpu.SemaphoreType.DMA,
            pltpu.SemaphoreType.DMA,
        ],
        compiler_params=pltpu.CompilerParams(
            needs_layout_passes=False, use_tc_tiling_on_sc=False),
    )
    return f(src, table, gamma, beta)


def kernel(src, word_embedding, ln_gamma, ln_beta):
    return _sc_lookup_ln(src.astype(jnp.int32), word_embedding, ln_gamma, ln_beta)
